# Initial kernel scaffold; baseline (speedup 1.0000x reference)
import jax
import jax.numpy as jnp
from jax.experimental import pallas as pl


def kernel(x, edge_index, W0, W1, b1, W2, b2):
    raise NotImplementedError("write your pallas kernel here")


# speedup vs baseline: 7.2989x; 7.2989x over previous
import functools

import jax
import jax.numpy as jnp
from jax import lax
from jax.experimental import pallas as pl
from jax.experimental.pallas import tpu as pltpu
from jax.experimental.pallas import tpu_sc as plsc

N = 10000
E = 320000
D = 128

NC = 2
NS = 16
NW = NC * NS

EPT = E // NW
EC = 80
NCHUNK = EPT // EC
NPAD = 10240
NH = NPAD // 2
TRASH = NH
RPT = NH // NS
DRPT = NPAD // NS

_MESH = plsc.VectorSubcoreMesh(core_axis_name="c", subcore_axis_name="s")


@functools.partial(
    pl.kernel,
    mesh=_MESH,
    out_type=jax.ShapeDtypeStruct((NC * NPAD,), jnp.float32),
    scratch_types=[
        pltpu.VMEM((EC,), jnp.int32),
        pltpu.VMEM((EC,), jnp.float32),
        pltpu.VMEM((DRPT,), jnp.float32),
        pltpu.VMEM_SHARED((NPAD,), jnp.float32),
    ],
)
def _sc_degree(dst_hbm, out_hbm, idx_v, ones_v, zbuf_v, acc_sh):
    c = lax.axis_index("c")
    s = lax.axis_index("s")
    wid = c * NS + s

    for i in range(EC // 16):
        ones_v[pl.ds(i * 16, 16)] = jnp.full((16,), 1.0, jnp.float32)

    def fill_zeros(i, _):
        zbuf_v[pl.ds(i * 16, 16)] = jnp.zeros((16,), jnp.float32)
        return 0

    lax.fori_loop(0, DRPT // 16, fill_zeros, 0)
    pltpu.sync_copy(zbuf_v, acc_sh.at[pl.ds(s * DRPT, DRPT)])
    plsc.subcore_barrier()

    def chunk(j, _):
        base = wid * EPT + j * EC
        pltpu.sync_copy(dst_hbm.at[pl.ds(base, EC)], idx_v)
        pltpu.sync_copy(ones_v, acc_sh.at[idx_v], add=True)
        return 0

    lax.fori_loop(0, NCHUNK, chunk, 0)
    plsc.subcore_barrier()
    pltpu.sync_copy(acc_sh.at[pl.ds(s * DRPT, DRPT)], zbuf_v)
    pltpu.sync_copy(zbuf_v, out_hbm.at[pl.ds(c * NPAD + s * DRPT, DRPT)])


@functools.partial(
    pl.kernel,
    mesh=_MESH,
    out_type=jax.ShapeDtypeStruct((NC * NPAD, D), jnp.float32),
    scratch_types=[
        pltpu.VMEM((EC,), jnp.int32),
        pltpu.VMEM((EC,), jnp.int32),
        pltpu.VMEM((EC, D), jnp.float32),
        pltpu.VMEM((RPT, D), jnp.float32),
        pltpu.VMEM_SHARED((NH + 8, D), jnp.float32),
        pltpu.SemaphoreType.DMA,
    ],
)
def _sc_edge_scatter(xws_hbm, src_hbm, dst0_hbm, dst1_hbm, zr_hbm, out_hbm,
                     isrc_v, idst_v, rows_v, zbuf_v, acc_sh, sem):
    c = lax.axis_index("c")
    s = lax.axis_index("s")
    wid = c * NS + s

    for half, dsth_hbm in ((0, dst0_hbm), (1, dst1_hbm)):
        lo = half * NH
        pltpu.sync_copy(zr_hbm, zbuf_v)
        pltpu.sync_copy(zbuf_v, acc_sh.at[pl.ds(s * RPT, RPT)])
        plsc.subcore_barrier()

        def chunk(j, _, dsth_hbm=dsth_hbm):
            base = wid * EPT + j * EC
            pltpu.sync_copy(src_hbm.at[pl.ds(base, EC)], isrc_v)
            pltpu.sync_copy(dsth_hbm.at[pl.ds(base, EC)], idst_v)
            pltpu.async_copy(xws_hbm.at[isrc_v], rows_v, sem).wait()
            pltpu.sync_copy(rows_v, acc_sh.at[idst_v], add=True)
            return 0

        lax.fori_loop(0, NCHUNK, chunk, 0)
        plsc.subcore_barrier()
        pltpu.sync_copy(acc_sh.at[pl.ds(s * RPT, RPT)], zbuf_v)
        pltpu.sync_copy(zbuf_v, out_hbm.at[pl.ds(c * NPAD + lo + s * RPT, RPT)])
        plsc.subcore_barrier()


RB = 1024
GRID = NPAD // RB
EROWS = E // 128


def _tc_remap(dst2d):
    def body(d_ref, o0_ref, o1_ref):
        d = d_ref[...]
        o0_ref[...] = jnp.where(d < NH, d, TRASH)
        o1_ref[...] = jnp.where(d >= NH, d - NH, TRASH)

    return pl.pallas_call(
        body,
        out_shape=[
            jax.ShapeDtypeStruct((EROWS, 128), jnp.int32),
            jax.ShapeDtypeStruct((EROWS, 128), jnp.int32),
        ],
    )(dst2d)


def _tc_dinv(degp):
    def body(degp_ref, o_ref):
        deg = degp_ref[0] + degp_ref[1] + 1.0
        dinv = lax.rsqrt(deg)
        o_ref[...] = jnp.broadcast_to(dinv[:, None], (RB, D))

    return pl.pallas_call(
        body,
        grid=(GRID,),
        in_specs=[pl.BlockSpec((NC, RB), lambda i: (0, i))],
        out_specs=pl.BlockSpec((RB, D), lambda i: (i, 0)),
        out_shape=jax.ShapeDtypeStruct((NPAD, D), jnp.float32),
    )(degp)


def _tc_first(x, W0, W1, dinvb):
    def body(x_ref, w0_ref, w1_ref, dinv_ref, o_ref):
        h = jnp.dot(x_ref[...], w0_ref[...], preferred_element_type=jnp.float32)
        xw = jnp.dot(h, w1_ref[...], preferred_element_type=jnp.float32)
        o_ref[...] = xw * dinv_ref[...]

    return pl.pallas_call(
        body,
        grid=(GRID,),
        in_specs=[
            pl.BlockSpec((RB, D), lambda i: (i, 0)),
            pl.BlockSpec((D, D), lambda i: (0, 0)),
            pl.BlockSpec((D, D), lambda i: (0, 0)),
            pl.BlockSpec((RB, D), lambda i: (i, 0)),
        ],
        out_specs=pl.BlockSpec((RB, D), lambda i: (i, 0)),
        out_shape=jax.ShapeDtypeStruct((NPAD, D), jnp.float32),
    )(x, W0, W1, dinvb)


def _tc_mid(p, xws1, dinvb, b1, W2):
    def body(p_ref, xws_ref, dinv_ref, b_ref, w2_ref, o_ref):
        dinv = dinv_ref[...]
        tot = p_ref[0] + p_ref[1] + xws_ref[...]
        h1 = jnp.maximum(tot * dinv + b_ref[...], 0.0)
        xw2 = jnp.dot(h1, w2_ref[...], preferred_element_type=jnp.float32)
        o_ref[...] = xw2 * dinv

    return pl.pallas_call(
        body,
        grid=(GRID,),
        in_specs=[
            pl.BlockSpec((NC, RB, D), lambda i: (0, i, 0)),
            pl.BlockSpec((RB, D), lambda i: (i, 0)),
            pl.BlockSpec((RB, D), lambda i: (i, 0)),
            pl.BlockSpec((1, D), lambda i: (0, 0)),
            pl.BlockSpec((D, D), lambda i: (0, 0)),
        ],
        out_specs=pl.BlockSpec((RB, D), lambda i: (i, 0)),
        out_shape=jax.ShapeDtypeStruct((NPAD, D), jnp.float32),
    )(p, xws1, dinvb, b1, W2)


def _tc_last(p, xws2, dinvb, b2):
    def body(p_ref, xws_ref, dinv_ref, b_ref, o_ref):
        tot = p_ref[0] + p_ref[1] + xws_ref[...]
        o_ref[...] = tot * dinv_ref[...] + b_ref[...]

    return pl.pallas_call(
        body,
        grid=(GRID,),
        in_specs=[
            pl.BlockSpec((NC, RB, D), lambda i: (0, i, 0)),
            pl.BlockSpec((RB, D), lambda i: (i, 0)),
            pl.BlockSpec((RB, D), lambda i: (i, 0)),
            pl.BlockSpec((1, D), lambda i: (0, 0)),
        ],
        out_specs=pl.BlockSpec((RB, D), lambda i: (i, 0)),
        out_shape=jax.ShapeDtypeStruct((NPAD, D), jnp.float32),
    )(p, xws2, dinvb, b2)


def kernel(x, edge_index, W0, W1, b1, W2, b2):
    src = edge_index[0].astype(jnp.int32)
    dst = edge_index[1].astype(jnp.int32)
    d0, d1 = _tc_remap(dst.reshape(EROWS, 128))
    dst0 = d0.reshape(E)
    dst1 = d1.reshape(E)
    xpad = jnp.zeros((NPAD, D), jnp.float32).at[:N].set(x)
    zrows = jnp.zeros((RPT, D), jnp.float32)
    b1r = b1.reshape(1, D)
    b2r = b2.reshape(1, D)

    degp = _sc_degree(dst).reshape(NC, NPAD)
    dinvb = _tc_dinv(degp)
    xws1 = _tc_first(xpad, W0, W1, dinvb)
    p1 = _sc_edge_scatter(xws1, src, dst0, dst1, zrows)
    p1 = p1.reshape(NC, NPAD, D)
    xws2 = _tc_mid(p1, xws1, dinvb, b1r, W2)
    p2 = _sc_edge_scatter(xws2, src, dst0, dst1, zrows)
    p2 = p2.reshape(NC, NPAD, D)
    return _tc_last(p2, xws2, dinvb, b2r)[:N]


# speedup vs baseline: 11.3951x; 1.5612x over previous
import functools

import jax
import jax.numpy as jnp
from jax import lax
from jax.experimental import pallas as pl
from jax.experimental.pallas import tpu as pltpu
from jax.experimental.pallas import tpu_sc as plsc

N = 10000
E = 320000
D = 128

NC = 2
NS = 16
NW = NC * NS

EPT = E // NW
EC = 80
NCHUNK = EPT // EC
NPAD = 10240
NH = NPAD // 2
TRASH = NH
RPT = NH // NS
DRPT = NPAD // NS

_MESH = plsc.VectorSubcoreMesh(core_axis_name="c", subcore_axis_name="s")


@functools.partial(
    pl.kernel,
    mesh=_MESH,
    out_type=jax.ShapeDtypeStruct((NC * NPAD,), jnp.float32),
    scratch_types=[
        pltpu.VMEM((EC,), jnp.int32),
        pltpu.VMEM((EC,), jnp.float32),
        pltpu.VMEM((DRPT,), jnp.float32),
        pltpu.VMEM_SHARED((NPAD,), jnp.float32),
    ],
)
def _sc_degree(dst_hbm, out_hbm, idx_v, ones_v, zbuf_v, acc_sh):
    c = lax.axis_index("c")
    s = lax.axis_index("s")
    wid = c * NS + s

    for i in range(EC // 16):
        ones_v[pl.ds(i * 16, 16)] = jnp.full((16,), 1.0, jnp.float32)

    def fill_zeros(i, _):
        zbuf_v[pl.ds(i * 16, 16)] = jnp.zeros((16,), jnp.float32)
        return 0

    lax.fori_loop(0, DRPT // 16, fill_zeros, 0)
    pltpu.sync_copy(zbuf_v, acc_sh.at[pl.ds(s * DRPT, DRPT)])
    plsc.subcore_barrier()

    def chunk(j, _):
        base = wid * EPT + j * EC
        pltpu.sync_copy(dst_hbm.at[pl.ds(base, EC)], idx_v)
        pltpu.sync_copy(ones_v, acc_sh.at[idx_v], add=True)
        return 0

    lax.fori_loop(0, NCHUNK, chunk, 0)
    plsc.subcore_barrier()
    pltpu.sync_copy(acc_sh.at[pl.ds(s * DRPT, DRPT)], zbuf_v)
    pltpu.sync_copy(zbuf_v, out_hbm.at[pl.ds(c * NPAD + s * DRPT, DRPT)])


@functools.partial(
    pl.kernel,
    mesh=_MESH,
    out_type=jax.ShapeDtypeStruct((NC * NPAD, D), jnp.float32),
    scratch_types=[
        pltpu.VMEM((EC,), jnp.int32),
        pltpu.VMEM((EC,), jnp.int32),
        pltpu.VMEM((EC, D), jnp.float32),
        pltpu.VMEM((EC,), jnp.int32),
        pltpu.VMEM((EC,), jnp.int32),
        pltpu.VMEM((EC, D), jnp.float32),
        pltpu.VMEM((RPT, D), jnp.float32),
        pltpu.VMEM_SHARED((NH + 8, D), jnp.float32),
        pltpu.SemaphoreType.DMA,
        pltpu.SemaphoreType.DMA,
    ],
)
def _sc_edge_scatter(xws_hbm, src_hbm, dst0_hbm, dst1_hbm, zr_hbm, out_hbm,
                     isrc_a, idst_a, rows_a, isrc_b, idst_b, rows_b,
                     zbuf_v, acc_sh, sem_a, sem_b):
    c = lax.axis_index("c")
    s = lax.axis_index("s")
    wid = c * NS + s

    for half, dsth_hbm in ((0, dst0_hbm), (1, dst1_hbm)):
        lo = half * NH
        pltpu.sync_copy(zr_hbm, zbuf_v)
        pltpu.sync_copy(zbuf_v, acc_sh.at[pl.ds(s * RPT, RPT)])
        plsc.subcore_barrier()

        ebase = wid * EPT

        def load_and_gather(cidx, isrc_v, idst_v, rows_v, sem,
                            dsth_hbm=dsth_hbm):
            base = ebase + cidx * EC
            pltpu.sync_copy(src_hbm.at[pl.ds(base, EC)], isrc_v)
            pltpu.sync_copy(dsth_hbm.at[pl.ds(base, EC)], idst_v)
            return pltpu.async_copy(xws_hbm.at[isrc_v], rows_v, sem)

        load_and_gather(0, isrc_a, idst_a, rows_a, sem_a)

        def pair(k, _):
            load_and_gather(2 * k + 1, isrc_b, idst_b, rows_b, sem_b)
            pltpu.make_async_copy(xws_hbm.at[isrc_a], rows_a, sem_a).wait()
            pltpu.sync_copy(rows_a, acc_sh.at[idst_a], add=True)
            load_and_gather(2 * k + 2, isrc_a, idst_a, rows_a, sem_a)
            pltpu.make_async_copy(xws_hbm.at[isrc_b], rows_b, sem_b).wait()
            pltpu.sync_copy(rows_b, acc_sh.at[idst_b], add=True)
            return 0

        lax.fori_loop(0, (NCHUNK - 1) // 2, pair, 0)
        pltpu.make_async_copy(xws_hbm.at[isrc_a], rows_a, sem_a).wait()
        pltpu.sync_copy(rows_a, acc_sh.at[idst_a], add=True)

        plsc.subcore_barrier()
        pltpu.sync_copy(acc_sh.at[pl.ds(s * RPT, RPT)], zbuf_v)
        pltpu.sync_copy(zbuf_v, out_hbm.at[pl.ds(c * NPAD + lo + s * RPT, RPT)])
        plsc.subcore_barrier()


RB = 1024
GRID = NPAD // RB
EROWS = E // 128


def _tc_remap(dst2d):
    def body(d_ref, o0_ref, o1_ref):
        d = d_ref[...]
        o0_ref[...] = jnp.where(d < NH, d, TRASH)
        o1_ref[...] = jnp.where(d >= NH, d - NH, TRASH)

    return pl.pallas_call(
        body,
        out_shape=[
            jax.ShapeDtypeStruct((EROWS, 128), jnp.int32),
            jax.ShapeDtypeStruct((EROWS, 128), jnp.int32),
        ],
    )(dst2d)


def _tc_dinv(degp):
    def body(degp_ref, o_ref):
        deg = degp_ref[0] + degp_ref[1] + 1.0
        dinv = lax.rsqrt(deg)
        o_ref[...] = jnp.broadcast_to(dinv[:, None], (RB, D))

    return pl.pallas_call(
        body,
        grid=(GRID,),
        in_specs=[pl.BlockSpec((NC, RB), lambda i: (0, i))],
        out_specs=pl.BlockSpec((RB, D), lambda i: (i, 0)),
        out_shape=jax.ShapeDtypeStruct((NPAD, D), jnp.float32),
    )(degp)


def _tc_first(x, W0, W1, dinvb):
    def body(x_ref, w0_ref, w1_ref, dinv_ref, o_ref):
        h = jnp.dot(x_ref[...], w0_ref[...], preferred_element_type=jnp.float32)
        xw = jnp.dot(h, w1_ref[...], preferred_element_type=jnp.float32)
        o_ref[...] = xw * dinv_ref[...]

    return pl.pallas_call(
        body,
        grid=(GRID,),
        in_specs=[
            pl.BlockSpec((RB, D), lambda i: (i, 0)),
            pl.BlockSpec((D, D), lambda i: (0, 0)),
            pl.BlockSpec((D, D), lambda i: (0, 0)),
            pl.BlockSpec((RB, D), lambda i: (i, 0)),
        ],
        out_specs=pl.BlockSpec((RB, D), lambda i: (i, 0)),
        out_shape=jax.ShapeDtypeStruct((NPAD, D), jnp.float32),
    )(x, W0, W1, dinvb)


def _tc_mid(p, xws1, dinvb, b1, W2):
    def body(p_ref, xws_ref, dinv_ref, b_ref, w2_ref, o_ref):
        dinv = dinv_ref[...]
        tot = p_ref[0] + p_ref[1] + xws_ref[...]
        h1 = jnp.maximum(tot * dinv + b_ref[...], 0.0)
        xw2 = jnp.dot(h1, w2_ref[...], preferred_element_type=jnp.float32)
        o_ref[...] = xw2 * dinv

    return pl.pallas_call(
        body,
        grid=(GRID,),
        in_specs=[
            pl.BlockSpec((NC, RB, D), lambda i: (0, i, 0)),
            pl.BlockSpec((RB, D), lambda i: (i, 0)),
            pl.BlockSpec((RB, D), lambda i: (i, 0)),
            pl.BlockSpec((1, D), lambda i: (0, 0)),
            pl.BlockSpec((D, D), lambda i: (0, 0)),
        ],
        out_specs=pl.BlockSpec((RB, D), lambda i: (i, 0)),
        out_shape=jax.ShapeDtypeStruct((NPAD, D), jnp.float32),
    )(p, xws1, dinvb, b1, W2)


def _tc_last(p, xws2, dinvb, b2):
    def body(p_ref, xws_ref, dinv_ref, b_ref, o_ref):
        tot = p_ref[0] + p_ref[1] + xws_ref[...]
        o_ref[...] = tot * dinv_ref[...] + b_ref[...]

    return pl.pallas_call(
        body,
        grid=(GRID,),
        in_specs=[
            pl.BlockSpec((NC, RB, D), lambda i: (0, i, 0)),
            pl.BlockSpec((RB, D), lambda i: (i, 0)),
            pl.BlockSpec((RB, D), lambda i: (i, 0)),
            pl.BlockSpec((1, D), lambda i: (0, 0)),
        ],
        out_specs=pl.BlockSpec((RB, D), lambda i: (i, 0)),
        out_shape=jax.ShapeDtypeStruct((NPAD, D), jnp.float32),
    )(p, xws2, dinvb, b2)


def kernel(x, edge_index, W0, W1, b1, W2, b2):
    src = edge_index[0].astype(jnp.int32)
    dst = edge_index[1].astype(jnp.int32)
    d0, d1 = _tc_remap(dst.reshape(EROWS, 128))
    dst0 = d0.reshape(E)
    dst1 = d1.reshape(E)
    xpad = jnp.zeros((NPAD, D), jnp.float32).at[:N].set(x)
    zrows = jnp.zeros((RPT, D), jnp.float32)
    b1r = b1.reshape(1, D)
    b2r = b2.reshape(1, D)

    degp = _sc_degree(dst).reshape(NC, NPAD)
    dinvb = _tc_dinv(degp)
    xws1 = _tc_first(xpad, W0, W1, dinvb)
    p1 = _sc_edge_scatter(xws1, src, dst0, dst1, zrows)
    p1 = p1.reshape(NC, NPAD, D)
    xws2 = _tc_mid(p1, xws1, dinvb, b1r, W2)
    p2 = _sc_edge_scatter(xws2, src, dst0, dst1, zrows)
    p2 = p2.reshape(NC, NPAD, D)
    return _tc_last(p2, xws2, dinvb, b2r)[:N]
